# baseline (speedup 1.0000x reference)
import math

import jax
import jax.numpy as jnp
from jax.experimental import pallas as pl

D_MODEL = 2048
HIDDEN = 128
N_EXPERTS = 16
CAPACITY_FACTOR = 2.0


KC = 256
NK = D_MODEL // KC


def _logits_body(x_ref, w1_ref, w2_ref, out_ref, h_ref):
    k = pl.program_id(1)

    @pl.when(k == 0)
    def _():
        h_ref[...] = jnp.zeros_like(h_ref)

    h_ref[...] += jnp.dot(
        x_ref[...], w1_ref[...], preferred_element_type=jnp.float32
    )

    @pl.when(k == NK - 1)
    def _():
        h = h_ref[...]
        h = h * jax.nn.sigmoid(h)
        out_ref[...] = jnp.dot(h, w2_ref[...], preferred_element_type=jnp.float32)


def _router_logits(x):
    from jax.experimental.pallas import tpu as pltpu

    M = x.shape[0]
    BM = 512
    return pl.pallas_call(
        _logits_body,
        grid=(M // BM, NK),
        in_specs=[
            pl.BlockSpec((BM, KC), lambda i, k: (i, k)),
            pl.BlockSpec((KC, HIDDEN), lambda i, k: (k, 0)),
            pl.BlockSpec((HIDDEN, N_EXPERTS), lambda i, k: (0, 0)),
        ],
        out_specs=pl.BlockSpec((BM, N_EXPERTS), lambda i, k: (i, 0)),
        out_shape=jax.ShapeDtypeStruct((M, N_EXPERTS), jnp.float32),
        scratch_shapes=[pltpu.VMEM((BM, HIDDEN), jnp.float32)],
    )


def kernel(hidden_states, W1, W2):
    batch, seq_len, d_model = hidden_states.shape
    capacity = int(math.ceil(seq_len * CAPACITY_FACTOR / N_EXPERTS))
    x = hidden_states.reshape(batch * seq_len, d_model)
    router_logits = _router_logits(x)(x, W1, W2).reshape(batch, seq_len, N_EXPERTS)

    expert_logits = jnp.transpose(router_logits, (0, 2, 1))
    expert_probs = jax.nn.softmax(expert_logits, axis=-1)
    expert_weights, token_indices = jax.lax.top_k(expert_probs, capacity)
    expert_weights = expert_weights / (
        jnp.sum(expert_weights, axis=-1, keepdims=True) + 1e-9
    )
    return (expert_weights, token_indices, router_logits, capacity)
